# Initial kernel scaffold; baseline (speedup 1.0000x reference)
import functools

import jax
import jax.numpy as jnp
from jax import lax
from jax.experimental import pallas as pl
from jax.experimental.pallas import tpu as pltpu
from jax.experimental.pallas import tpu_sc as plsc

N = 10000
E = 320000
D = 128

NW = 32
G = 32
NE = E + N
GROUPS = -(-NE // (NW * G))
C = GROUPS * G
TOTAL = NW * C
NP_ = 10112
JUNK = N
RPT = NP_ // 16


def _tc_prep(x, W, att2):
    def body(x_ref, w_ref, a_ref, h_ref, aux_ref):
        h = jnp.dot(x_ref[...], w_ref[...],
                    preferred_element_type=jnp.float32,
                    precision=lax.Precision.HIGHEST)
        h_ref[...] = h
        aux = lax.dot_general(a_ref[...], h, (((1,), (1,)), ((), ())),
                              preferred_element_type=jnp.float32,
                              precision=lax.Precision.HIGHEST)
        aux_ref[...] = jnp.concatenate(
            [aux, jnp.zeros((2, NP_ - N), jnp.float32)], axis=1)

    return pl.pallas_call(
        body,
        out_shape=(jax.ShapeDtypeStruct((N, D), jnp.float32),
                   jax.ShapeDtypeStruct((2, NP_), jnp.float32)),
    )(x, W, att2)


def _sc_edges(h, aux, src_t, dst_t):
    mesh = plsc.VectorSubcoreMesh(core_axis_name="c", subcore_axis_name="s")

    @functools.partial(
        pl.kernel,
        out_type=(jax.ShapeDtypeStruct((2, NP_, D), jnp.float32),
                  jax.ShapeDtypeStruct((NW, NP_), jnp.float32)),
        mesh=mesh,
        scratch_types=[
            pltpu.VMEM((NP_,), jnp.float32),
            pltpu.VMEM((NP_,), jnp.float32),
            pltpu.VMEM((GROUPS, G), jnp.int32),
            pltpu.VMEM((GROUPS, G), jnp.int32),
            pltpu.VMEM((NP_,), jnp.float32),
            pltpu.VMEM((G, D), jnp.float32),
            pltpu.VMEM((G,), jnp.float32),
            pltpu.VMEM_SHARED((NP_, D), jnp.float32),
        ],
    )
    def k(h_hbm, aux_hbm, src_hbm, dst_hbm, acc_hbm, den_hbm,
          asrc_v, adst_v, src_v, dst_v, den_v, rows_v, p_v, acc_sh):
        c = lax.axis_index("c")
        s = lax.axis_index("s")
        wid = c * 16 + s
        zv = jnp.zeros((16,), jnp.float32)

        @pl.loop(0, NP_ // 16)
        def _(i):
            den_v[pl.ds(pl.multiple_of(i * 16, 8), 16)] = zv

        @pl.loop(0, G)
        def _(r):
            for j in range(8):
                rows_v[r, pl.ds(j * 16, 16)] = zv

        @pl.loop(0, RPT // 8)
        def _(i):
            pltpu.sync_copy(
                rows_v.at[pl.ds(0, 8)],
                acc_sh.at[pl.ds(s * RPT + i * 8, 8)])

        pltpu.sync_copy(aux_hbm.at[0], asrc_v)
        pltpu.sync_copy(aux_hbm.at[1], adst_v)
        pltpu.sync_copy(src_hbm.at[wid], src_v)
        pltpu.sync_copy(dst_hbm.at[wid], dst_v)
        plsc.subcore_barrier()

        @pl.loop(0, GROUPS)
        def _(g):
            pltpu.sync_copy(h_hbm.at[src_v.at[g]], rows_v)
            for j in range(G // 16):
                sv = src_v[g, pl.ds(j * 16, 16)]
                dv = dst_v[g, pl.ds(j * 16, 16)]
                e = (plsc.load_gather(asrc_v, [sv])
                     + plsc.load_gather(adst_v, [dv]))
                e = jnp.where(e >= 0.0, e, 0.2 * e)
                p = jnp.exp(e)
                plsc.addupdate_scatter(den_v, [dv], p)
                p_v[pl.ds(j * 16, 16)] = p
            for ei in range(G):
                av = plsc.load_gather(
                    p_v, [jnp.full((16,), ei, jnp.int32)])
                for j in range(8):
                    r = rows_v[ei, pl.ds(j * 16, 16)]
                    rows_v[ei, pl.ds(j * 16, 16)] = r * av
            pltpu.sync_copy(rows_v, acc_sh.at[dst_v.at[g]], add=True)

        plsc.subcore_barrier()
        pltpu.sync_copy(acc_sh.at[pl.ds(s * RPT, RPT)],
                        acc_hbm.at[c, pl.ds(s * RPT, RPT)])
        pltpu.sync_copy(den_v, den_hbm.at[wid])

    return k(h, aux, src_t, dst_t)


def _tc_final(acc, den, bias2):
    def body(acc_ref, den_ref, b_ref, o_ref):
        a = acc_ref[0] + acc_ref[1]
        dsum = jnp.sum(den_ref[...], axis=0)
        o_ref[...] = jnp.tanh(
            a[:N] / (dsum[:N, None] + 1e-16) + b_ref[...])

    return pl.pallas_call(
        body,
        out_shape=jax.ShapeDtypeStruct((N, D), jnp.float32),
    )(acc, den, bias2)


def kernel(x, edge_index, W, att_src, att_dst, bias):
    src = edge_index[0].astype(jnp.int32)
    dst = edge_index[1].astype(jnp.int32)
    loop = jnp.arange(N, dtype=jnp.int32)
    pad = TOTAL - NE
    src_all = jnp.concatenate(
        [src, loop, jnp.zeros((pad,), jnp.int32)])
    dst_all = jnp.concatenate(
        [dst, loop, jnp.full((pad,), JUNK, jnp.int32)])
    src_t = src_all.reshape(NW, GROUPS, G)
    dst_t = dst_all.reshape(NW, GROUPS, G)
    att2 = jnp.stack([att_src, att_dst]).astype(jnp.float32)

    h, aux = _tc_prep(x.astype(jnp.float32), W.astype(jnp.float32), att2)
    acc, den = _sc_edges(h, aux, src_t, dst_t)
    return _tc_final(acc, den, bias.astype(jnp.float32).reshape(1, D))


# speedup vs baseline: 229.6440x; 229.6440x over previous
import dataclasses
import functools

import jax
import jax.numpy as jnp
from jax import lax
from jax.experimental import pallas as pl
from jax.experimental.pallas import tpu as pltpu
from jax.experimental.pallas import tpu_sc as plsc

N = 10000
E = 320000
D = 128

NW = 32
G = 128
NE = E + N
GROUPS = -(-NE // (NW * G))
TOTAL = NW * GROUPS * G
NT = 10112
NA = 10112
JUNK = N
RPT = NA // 16


def _loop(n):
    return pl.loop(jnp.int32(0), jnp.int32(n))


def _tc_prep(x, W, att2):
    def body(x_ref, w_ref, a_ref, h_ref, aux_ref):
        h = jnp.dot(x_ref[...], w_ref[...],
                    preferred_element_type=jnp.float32,
                    precision=lax.Precision.HIGHEST)
        h_ref[...] = h
        aux = lax.dot_general(a_ref[...], h, (((1,), (1,)), ((), ())),
                              preferred_element_type=jnp.float32,
                              precision=lax.Precision.HIGHEST)
        aux_ref[...] = jnp.concatenate(
            [aux, jnp.zeros((2, NT - N), jnp.float32)], axis=1)

    return pl.pallas_call(
        body,
        out_shape=(jax.ShapeDtypeStruct((N, D), jnp.float32),
                   jax.ShapeDtypeStruct((2, NT), jnp.float32)),
    )(x, W, att2)


def _sc_edges(h, aux, src_t, dst_t):
    mesh = plsc.VectorSubcoreMesh(core_axis_name="c", subcore_axis_name="s",
                                  num_cores=2, num_subcores=16)
    cp = pltpu.CompilerParams()
    if "needs_layout_passes" in pltpu.CompilerParams.__dataclass_fields__:
        cp = dataclasses.replace(cp, needs_layout_passes=False)

    @functools.partial(
        pl.kernel,
        out_type=(jax.ShapeDtypeStruct((2, NA, D), jnp.float32),
                  jax.ShapeDtypeStruct((NW, NA), jnp.float32)),
        mesh=mesh,
        scratch_types=[
            pltpu.VMEM((NT,), jnp.float32),
            pltpu.VMEM((NT,), jnp.float32),
            pltpu.VMEM((G,), jnp.int32),
            pltpu.VMEM((1, G), jnp.int32),
            pltpu.VMEM((NA,), jnp.float32),
            pltpu.VMEM((G, D), jnp.float32),
            pltpu.VMEM((G,), jnp.float32),
            pltpu.VMEM_SHARED((NA, D), jnp.float32),
        ],
        compiler_params=cp,
    )
    def k(h_hbm, aux_hbm, src_hbm, dst_hbm, acc_hbm, den_hbm,
          asrc_v, adst_v, src_v, dst_v, den_v, rows_v, p_v, acc_sh):
        i32 = jnp.int32
        c = lax.axis_index("c").astype(i32)
        s = lax.axis_index("s").astype(i32)
        wid = c * i32(16) + s
        zv = jnp.zeros((16,), jnp.float32)

        @_loop(NA // 16)
        def _(i):
            den_v[pl.ds(pl.multiple_of(i * i32(16), 8), 16)] = zv

        @_loop(G)
        def _(r):
            for j in range(8):
                rows_v[r, pl.ds(j * 16, 16)] = zv

        @_loop(4)
        def _(i):
            pltpu.sync_copy(
                rows_v,
                acc_sh.at[pl.ds(s * i32(RPT) + i * i32(G), G)])
        pltpu.sync_copy(
            rows_v.at[pl.ds(0, RPT - 4 * G)],
            acc_sh.at[pl.ds(s * i32(RPT) + i32(4 * G), RPT - 4 * G)])

        pltpu.sync_copy(aux_hbm.at[i32(0)], asrc_v)
        pltpu.sync_copy(aux_hbm.at[i32(1)], adst_v)
        plsc.subcore_barrier()

        @_loop(GROUPS)
        def _(g):
            pltpu.sync_copy(src_hbm.at[wid, g], src_v)
            pltpu.sync_copy(dst_hbm.at[wid, g], dst_v.at[i32(0)])
            pltpu.sync_copy(h_hbm.at[src_v], rows_v)
            for j in range(G // 16):
                sv = src_v[pl.ds(j * 16, 16)]
                dv = dst_v[0, pl.ds(j * 16, 16)]
                e = (plsc.load_gather(asrc_v, [sv])
                     + plsc.load_gather(adst_v, [dv]))
                e = jnp.where(e >= 0.0, e, 0.2 * e)
                p = jnp.exp(e)
                plsc.addupdate_scatter(den_v, [dv], p)
                p_v[pl.ds(j * 16, 16)] = p

            @_loop(G)
            def _(ei):
                av = plsc.load_gather(p_v, [jnp.full((16,), ei, i32)])
                for j in range(8):
                    r = rows_v[ei, pl.ds(j * 16, 16)]
                    rows_v[ei, pl.ds(j * 16, 16)] = r * av

            pltpu.sync_copy(rows_v, acc_sh.at[dst_v.at[i32(0)]], add=True)

        plsc.subcore_barrier()
        pltpu.sync_copy(acc_sh.at[pl.ds(s * i32(RPT), RPT)],
                        acc_hbm.at[c, pl.ds(s * i32(RPT), RPT)])
        pltpu.sync_copy(den_v, den_hbm.at[wid])

    return k(h, aux, src_t, dst_t)


def _tc_final(acc, den, bias2):
    def body(acc_ref, den_ref, b_ref, o_ref):
        a = acc_ref[0] + acc_ref[1]
        dsum = jnp.sum(den_ref[...], axis=0)
        o_ref[...] = jnp.tanh(
            a[:N] / (dsum[:N, None] + 1e-16) + b_ref[...])

    return pl.pallas_call(
        body,
        out_shape=jax.ShapeDtypeStruct((N, D), jnp.float32),
    )(acc, den, bias2)


def kernel(x, edge_index, W, att_src, att_dst, bias):
    src = edge_index[0].astype(jnp.int32)
    dst = edge_index[1].astype(jnp.int32)
    loop = jnp.arange(N, dtype=jnp.int32)
    pad = TOTAL - NE
    src_all = jnp.concatenate(
        [src, loop, jnp.zeros((pad,), jnp.int32)])
    dst_all = jnp.concatenate(
        [dst, loop, jnp.full((pad,), JUNK, jnp.int32)])
    src_t = src_all.reshape(NW, GROUPS, G)
    dst_t = dst_all.reshape(NW, GROUPS, G)
    att2 = jnp.stack([att_src, att_dst]).astype(jnp.float32)

    h, aux = _tc_prep(x.astype(jnp.float32), W.astype(jnp.float32), att2)
    acc, den = _sc_edges(h, aux, src_t, dst_t)
    out = _tc_final(acc, den, bias.astype(jnp.float32).reshape(1, D))
    return out.astype(jnp.result_type(x.dtype, W.dtype))


# speedup vs baseline: 246.6496x; 1.0741x over previous
import dataclasses
import functools

import jax
import jax.numpy as jnp
from jax import lax
from jax.experimental import pallas as pl
from jax.experimental.pallas import tpu as pltpu
from jax.experimental.pallas import tpu_sc as plsc

N = 10000
E = 320000
D = 128

NW = 32
G = 128
NE = E + N
GROUPS = -(-NE // (NW * G))
TOTAL = NW * GROUPS * G
NT = 10112
NA = 10112
JUNK = N
RPT = NA // 16


def _loop(n):
    return pl.loop(jnp.int32(0), jnp.int32(n))


def _tc_prep(x, W, att2):
    def body(x_ref, w_ref, a_ref, h_ref, aux_ref):
        h = jnp.dot(x_ref[...], w_ref[...],
                    preferred_element_type=jnp.float32,
                    precision=lax.Precision.HIGHEST)
        h_ref[...] = h
        aux = lax.dot_general(a_ref[...], h, (((1,), (1,)), ((), ())),
                              preferred_element_type=jnp.float32,
                              precision=lax.Precision.HIGHEST)
        aux_ref[...] = jnp.concatenate(
            [aux, jnp.zeros((2, NT - N), jnp.float32)], axis=1)

    return pl.pallas_call(
        body,
        out_shape=(jax.ShapeDtypeStruct((N, D), jnp.float32),
                   jax.ShapeDtypeStruct((2, NT), jnp.float32)),
    )(x, W, att2)


def _sc_edges(h, aux, idx_t):
    mesh = plsc.VectorSubcoreMesh(core_axis_name="c", subcore_axis_name="s",
                                  num_cores=2, num_subcores=16)
    cp = pltpu.CompilerParams()
    if "needs_layout_passes" in pltpu.CompilerParams.__dataclass_fields__:
        cp = dataclasses.replace(cp, needs_layout_passes=False)

    @functools.partial(
        pl.kernel,
        out_type=(jax.ShapeDtypeStruct((2, NA, D), jnp.float32),
                  jax.ShapeDtypeStruct((NW, NA), jnp.float32)),
        mesh=mesh,
        scratch_types=[
            pltpu.VMEM((NT,), jnp.float32),
            pltpu.VMEM((NT,), jnp.float32),
            pltpu.VMEM((2, G), jnp.int32),
            pltpu.VMEM((NA,), jnp.float32),
            pltpu.VMEM((G, D), jnp.float32),
            pltpu.VMEM((G,), jnp.float32),
            pltpu.VMEM_SHARED((NA, D), jnp.float32),
            pltpu.SemaphoreType.DMA,
        ],
        compiler_params=cp,
    )
    def k(h_hbm, aux_hbm, idx_hbm, acc_hbm, den_hbm,
          asrc_v, adst_v, idx_v, den_v, rows_v, p_v, acc_sh, sem):
        i32 = jnp.int32
        c = lax.axis_index("c").astype(i32)
        s = lax.axis_index("s").astype(i32)
        wid = c * i32(16) + s
        zv = jnp.zeros((16,), jnp.float32)

        @_loop(NA // 16)
        def _(i):
            den_v[pl.ds(pl.multiple_of(i * i32(16), 8), 16)] = zv

        @_loop(G)
        def _(r):
            for j in range(8):
                rows_v[r, pl.ds(j * 16, 16)] = zv

        @_loop(4)
        def _(i):
            pltpu.sync_copy(
                rows_v,
                acc_sh.at[pl.ds(s * i32(RPT) + i * i32(G), G)])
        pltpu.sync_copy(
            rows_v.at[pl.ds(0, RPT - 4 * G)],
            acc_sh.at[pl.ds(s * i32(RPT) + i32(4 * G), RPT - 4 * G)])

        pltpu.sync_copy(aux_hbm.at[i32(0)], asrc_v)
        pltpu.sync_copy(aux_hbm.at[i32(1)], adst_v)
        plsc.subcore_barrier()

        @_loop(GROUPS)
        def _(g):
            pltpu.sync_copy(idx_hbm.at[wid, g], idx_v)
            cp_h = pltpu.async_copy(h_hbm.at[idx_v.at[i32(0)]], rows_v, sem)
            for j in range(G // 16):
                sv = idx_v[0, pl.ds(j * 16, 16)]
                dv = idx_v[1, pl.ds(j * 16, 16)]
                e = (plsc.load_gather(asrc_v, [sv])
                     + plsc.load_gather(adst_v, [dv]))
                e = jnp.where(e >= 0.0, e, 0.2 * e)
                p = jnp.exp(e)
                plsc.addupdate_scatter(den_v, [dv], p)
                p_v[pl.ds(j * 16, 16)] = p
            cp_h.wait()

            @_loop(G // 4)
            def _(eb):
                e4 = eb * i32(4)
                for u in range(4):
                    av = plsc.load_gather(
                        p_v, [jnp.full((16,), e4 + i32(u), i32)])
                    for j in range(8):
                        r = rows_v[e4 + i32(u), pl.ds(j * 16, 16)]
                        rows_v[e4 + i32(u), pl.ds(j * 16, 16)] = r * av

            pltpu.sync_copy(rows_v, acc_sh.at[idx_v.at[i32(1)]], add=True)

        plsc.subcore_barrier()
        pltpu.sync_copy(acc_sh.at[pl.ds(s * i32(RPT), RPT)],
                        acc_hbm.at[c, pl.ds(s * i32(RPT), RPT)])
        pltpu.sync_copy(den_v, den_hbm.at[wid])

    return k(h, aux, idx_t)


def _tc_final(acc, den, bias2):
    def body(acc_ref, den_ref, b_ref, o_ref):
        a = acc_ref[0] + acc_ref[1]
        dsum = jnp.sum(den_ref[...], axis=0)
        o_ref[...] = jnp.tanh(
            a[:N] / (dsum[:N, None] + 1e-16) + b_ref[...])

    return pl.pallas_call(
        body,
        out_shape=jax.ShapeDtypeStruct((N, D), jnp.float32),
    )(acc, den, bias2)


def kernel(x, edge_index, W, att_src, att_dst, bias):
    src = edge_index[0].astype(jnp.int32)
    dst = edge_index[1].astype(jnp.int32)
    loop = jnp.arange(N, dtype=jnp.int32)
    pad = TOTAL - NE
    src_all = jnp.concatenate(
        [src, loop, jnp.zeros((pad,), jnp.int32)])
    dst_all = jnp.concatenate(
        [dst, loop, jnp.full((pad,), JUNK, jnp.int32)])
    idx_t = jnp.concatenate(
        [src_all.reshape(NW, GROUPS, 1, G),
         dst_all.reshape(NW, GROUPS, 1, G)], axis=2)
    att2 = jnp.stack([att_src, att_dst]).astype(jnp.float32)

    h, aux = _tc_prep(x.astype(jnp.float32), W.astype(jnp.float32), att2)
    acc, den = _sc_edges(h, aux, idx_t)
    out = _tc_final(acc, den, bias.astype(jnp.float32).reshape(1, D))
    return out.astype(jnp.result_type(x.dtype, W.dtype))
